# Initial kernel scaffold; baseline (speedup 1.0000x reference)
import functools

import jax
import jax.numpy as jnp
from jax import lax
from jax.experimental import pallas as pl
from jax.experimental.pallas import tpu as pltpu
from jax.experimental.pallas import tpu_sc as plsc

N = 100000
D = 128
E = 128
S = 5

NC = 2
NS = 16
NW = NC * NS

CHUNK = 80
NUM_CHUNKS = N // CHUNK
LANES = 16


def _sc_gather_sum_body(neigh_t, features, out, idx_v, rows_v, acc_v, sem):
    wid = lax.axis_index("s") * NC + lax.axis_index("c")

    base_chunks = NUM_CHUNKS // NW
    rem = NUM_CHUNKS - base_chunks * NW
    n_my = base_chunks + jnp.where(wid < rem, 1, 0)

    def do_chunk(k, _):
        chunk = wid + k * NW
        base = chunk * CHUNK
        for j in range(S):
            pltpu.sync_copy(neigh_t.at[j, pl.ds(base, CHUNK)], idx_v.at[j])
        copies = []
        for j in range(S):
            copies.append(
                pltpu.async_copy(features.at[idx_v.at[j]], rows_v.at[j], sem))
        for c in copies:
            c.wait()
        def sum_row(n, _):
            for l in range(D // LANES):
                sl = pl.ds(l * LANES, LANES)
                v = rows_v[0, n, sl]
                for j in range(1, S):
                    v = v + rows_v[j, n, sl]
                acc_v[n, sl] = v
            return 0
        lax.fori_loop(0, CHUNK, sum_row, 0)
        pltpu.sync_copy(acc_v, out.at[pl.ds(base, CHUNK)])
        return 0

    lax.fori_loop(0, n_my, do_chunk, 0)


@jax.jit
def _sc_gather_sum(neigh_t, features):
    mesh = plsc.VectorSubcoreMesh(core_axis_name="c", subcore_axis_name="s")
    return pl.kernel(
        _sc_gather_sum_body,
        out_type=jax.ShapeDtypeStruct((N, D), jnp.float32),
        mesh=mesh,
        scratch_types=[
            pltpu.VMEM((S, CHUNK), jnp.int32),
            pltpu.VMEM((S, CHUNK, D), jnp.float32),
            pltpu.VMEM((CHUNK, D), jnp.float32),
            pltpu.SemaphoreType.DMA,
        ],
    )(neigh_t, features)


BN = 2048


def _mm_body(f_ref, g_ref, w1_ref, w2_ref, o_ref):
    acc = lax.dot_general(
        w1_ref[...], f_ref[...], (((1,), (1,)), ((), ())),
        preferred_element_type=jnp.float32)
    acc = acc + lax.dot_general(
        w2_ref[...], g_ref[...], (((1,), (1,)), ((), ())),
        preferred_element_type=jnp.float32)
    o_ref[...] = jnp.maximum(acc, 0.0)


@jax.jit
def _tc_matmul(features, nsum, w1, w2s):
    grid = pl.cdiv(N, BN)
    return pl.pallas_call(
        _mm_body,
        grid=(grid,),
        in_specs=[
            pl.BlockSpec((BN, D), lambda i: (i, 0)),
            pl.BlockSpec((BN, D), lambda i: (i, 0)),
            pl.BlockSpec((E, D), lambda i: (0, 0)),
            pl.BlockSpec((E, D), lambda i: (0, 0)),
        ],
        out_specs=pl.BlockSpec((E, BN), lambda i: (0, i)),
        out_shape=jax.ShapeDtypeStruct((E, N), jnp.float32),
    )(features, nsum, w1, w2s)


def kernel(nodes, features, neigh_indices, W):
    del nodes
    neigh_t = jnp.transpose(neigh_indices)
    w1 = W[:, :D]
    w2s = W[:, D:] * (1.0 / S)
    nsum = _sc_gather_sum(neigh_t, features)
    return _tc_matmul(features, nsum, w1, w2s)


# speedup vs baseline: 4.3215x; 4.3215x over previous
import functools

import jax
import jax.numpy as jnp
from jax import lax
from jax.experimental import pallas as pl
from jax.experimental.pallas import tpu as pltpu
from jax.experimental.pallas import tpu_sc as plsc

N = 100000
D = 128
E = 128
S = 5

NC = 2
NS = 16
NW = NC * NS

CHUNK = 80
NUM_CHUNKS = N // CHUNK
LANES = 16


def _sc_gather_sum_body(neigh_flat, features, out, idx_v, rows_v, acc_v, sem):
    wid = lax.axis_index("s") * NC + lax.axis_index("c")

    base_chunks = NUM_CHUNKS // NW
    rem = NUM_CHUNKS - base_chunks * NW
    n_my = base_chunks + jnp.where(wid < rem, 1, 0)

    def do_chunk(k, _):
        chunk = wid + k * NW
        base = chunk * CHUNK
        for j in range(S):
            pltpu.sync_copy(neigh_flat.at[pl.ds(j * N + base, CHUNK)],
                            idx_v.at[j])
        copies = []
        for j in range(S):
            copies.append(
                pltpu.async_copy(features.at[idx_v.at[j]], rows_v.at[j], sem))
        for c in copies:
            c.wait()
        def sum_row(n, _):
            for l in range(D // LANES):
                sl = pl.ds(l * LANES, LANES)
                v = rows_v[0, n, sl]
                for j in range(1, S):
                    v = v + rows_v[j, n, sl]
                acc_v[n, sl] = v
            return 0
        lax.fori_loop(0, CHUNK, sum_row, 0)
        pltpu.sync_copy(acc_v, out.at[pl.ds(base, CHUNK)])
        return 0

    lax.fori_loop(0, n_my, do_chunk, 0)


@jax.jit
def _sc_gather_sum(neigh_flat, features):
    mesh = plsc.VectorSubcoreMesh(core_axis_name="c", subcore_axis_name="s")
    return pl.kernel(
        _sc_gather_sum_body,
        out_type=jax.ShapeDtypeStruct((N, D), jnp.float32),
        mesh=mesh,
        scratch_types=[
            pltpu.VMEM((S, CHUNK), jnp.int32),
            pltpu.VMEM((S, CHUNK, D), jnp.float32),
            pltpu.VMEM((CHUNK, D), jnp.float32),
            pltpu.SemaphoreType.DMA,
        ],
    )(neigh_flat, features)


BN = 2048


def _mm_body(f_ref, g_ref, w1_ref, w2_ref, o_ref):
    acc = lax.dot_general(
        w1_ref[...], f_ref[...], (((1,), (1,)), ((), ())),
        preferred_element_type=jnp.float32)
    acc = acc + lax.dot_general(
        w2_ref[...], g_ref[...], (((1,), (1,)), ((), ())),
        preferred_element_type=jnp.float32)
    o_ref[...] = jnp.maximum(acc, 0.0)


@jax.jit
def _tc_matmul(features, nsum, w1, w2s):
    grid = pl.cdiv(N, BN)
    return pl.pallas_call(
        _mm_body,
        grid=(grid,),
        in_specs=[
            pl.BlockSpec((BN, D), lambda i: (i, 0)),
            pl.BlockSpec((BN, D), lambda i: (i, 0)),
            pl.BlockSpec((E, D), lambda i: (0, 0)),
            pl.BlockSpec((E, D), lambda i: (0, 0)),
        ],
        out_specs=pl.BlockSpec((E, BN), lambda i: (0, i)),
        out_shape=jax.ShapeDtypeStruct((E, N), jnp.float32),
    )(features, nsum, w1, w2s)


def kernel(nodes, features, neigh_indices, W):
    del nodes
    neigh_flat = jnp.transpose(neigh_indices).reshape(-1)
    w1 = W[:, :D]
    w2s = W[:, D:] * (1.0 / S)
    nsum = _sc_gather_sum(neigh_flat, features)
    return _tc_matmul(features, nsum, w1, w2s)


# speedup vs baseline: 5.4178x; 1.2537x over previous
import functools

import jax
import jax.numpy as jnp
from jax import lax
from jax.experimental import pallas as pl
from jax.experimental.pallas import tpu as pltpu
from jax.experimental.pallas import tpu_sc as plsc

N = 100000
D = 128
E = 128
S = 5

NC = 2
NS = 16
NW = NC * NS

CHUNK = 80
NUM_CHUNKS = N // CHUNK
BASE_CHUNKS = NUM_CHUNKS // NW
REM = NUM_CHUNKS - BASE_CHUNKS * NW
LANES = 16


def _sc_gather_sum_body(neigh_flat, features, out,
                        idx_a, idx_b, rows_a, rows_b, acc_v, gs_a, gs_b):
    wid = lax.axis_index("s") * NC + lax.axis_index("c")

    def stage(chunk, idxb, rowsb, gsem):
        base = chunk * CHUNK
        for j in range(S):
            pltpu.sync_copy(neigh_flat.at[pl.ds(j * N + base, CHUNK)],
                            idxb.at[j])
        return [pltpu.async_copy(features.at[idxb.at[j]], rowsb.at[j], gsem)
                for j in range(S)]

    def consume(chunk, copies, rowsb):
        for c in copies:
            c.wait()

        def sum_row(n, _):
            for l in range(D // LANES):
                sl = pl.ds(l * LANES, LANES)
                v = rowsb[0, n, sl]
                for j in range(1, S):
                    v = v + rowsb[j, n, sl]
                acc_v[n, sl] = v
            return 0
        lax.fori_loop(0, CHUNK, sum_row, 0)
        pltpu.sync_copy(acc_v, out.at[pl.ds(chunk * CHUNK, CHUNK)])

    def pair(i, _):
        c0 = wid + (2 * i) * NW
        c1 = wid + (2 * i + 1) * NW
        cps0 = stage(c0, idx_a, rows_a, gs_a)
        cps1 = stage(c1, idx_b, rows_b, gs_b)
        consume(c0, cps0, rows_a)
        consume(c1, cps1, rows_b)
        return 0

    lax.fori_loop(0, (BASE_CHUNKS - 1) // 2, pair, 0)

    c38 = wid + (BASE_CHUNKS - 1) * NW
    c39 = jnp.where(wid < REM, BASE_CHUNKS * NW + wid, c38)
    cps0 = stage(c38, idx_a, rows_a, gs_a)
    cps1 = stage(c39, idx_b, rows_b, gs_b)
    consume(c38, cps0, rows_a)
    consume(c39, cps1, rows_b)


@jax.jit
def _sc_gather_sum(neigh_flat, features):
    mesh = plsc.VectorSubcoreMesh(core_axis_name="c", subcore_axis_name="s")
    return pl.kernel(
        _sc_gather_sum_body,
        out_type=jax.ShapeDtypeStruct((N, D), jnp.float32),
        mesh=mesh,
        scratch_types=[
            pltpu.VMEM((S, CHUNK), jnp.int32),
            pltpu.VMEM((S, CHUNK), jnp.int32),
            pltpu.VMEM((S, CHUNK, D), jnp.float32),
            pltpu.VMEM((S, CHUNK, D), jnp.float32),
            pltpu.VMEM((CHUNK, D), jnp.float32),
            pltpu.SemaphoreType.DMA,
            pltpu.SemaphoreType.DMA,
        ],
    )(neigh_flat, features)


BN = 2048


def _mm_body(f_ref, g_ref, w1_ref, w2_ref, o_ref):
    acc = lax.dot_general(
        w1_ref[...], f_ref[...], (((1,), (1,)), ((), ())),
        preferred_element_type=jnp.float32)
    acc = acc + lax.dot_general(
        w2_ref[...], g_ref[...], (((1,), (1,)), ((), ())),
        preferred_element_type=jnp.float32)
    o_ref[...] = jnp.maximum(acc, 0.0)


@jax.jit
def _tc_matmul(features, nsum, w1, w2s):
    grid = pl.cdiv(N, BN)
    return pl.pallas_call(
        _mm_body,
        grid=(grid,),
        in_specs=[
            pl.BlockSpec((BN, D), lambda i: (i, 0)),
            pl.BlockSpec((BN, D), lambda i: (i, 0)),
            pl.BlockSpec((E, D), lambda i: (0, 0)),
            pl.BlockSpec((E, D), lambda i: (0, 0)),
        ],
        out_specs=pl.BlockSpec((E, BN), lambda i: (0, i)),
        out_shape=jax.ShapeDtypeStruct((E, N), jnp.float32),
    )(features, nsum, w1, w2s)


def kernel(nodes, features, neigh_indices, W):
    del nodes
    neigh_flat = jnp.transpose(neigh_indices).reshape(-1)
    w1 = W[:, :D]
    w2s = W[:, D:] * (1.0 / S)
    nsum = _sc_gather_sum(neigh_flat, features)
    return _tc_matmul(features, nsum, w1, w2s)
